# baseline (speedup 1.0000x reference)
import functools

import jax
import jax.numpy as jnp
from jax import lax
from jax.experimental import pallas as pl
from jax.experimental.pallas import tpu as pltpu
from jax.experimental.pallas import tpu_sc as plsc

B, L, D, V = 4096, 50, 32, 100001
LAM = 0.01

NC, NS, LN = 2, 16, 16
NW = NC * NS
BW = B // NW
RW = BW * L
CB = 2
CR = CB * L
NCHUNK = BW // CB
RPAD = 128
NBUF = 2
PITCH = D + 1

def _sc_body(table_hbm, idx_hbm, u_hbm, pred_hbm, gsq_hbm, usq_hbm,
             idx_v, u_v, stage_v, rows_v, pred_v, gsq_v, usq_v, *sems):
    wid = lax.axis_index("s") * NC + lax.axis_index("c")
    pltpu.sync_copy(idx_hbm.at[wid], idx_v)
    pltpu.sync_copy(u_hbm.at[pl.ds(wid * BW, BW)], u_v)

    lane = lax.iota(jnp.int32, 16)
    zero = jnp.zeros((16,), jnp.float32)

    def start(c, slot):
        pltpu.async_copy(table_hbm.at[idx_v.at[2 * c]],
                         stage_v.at[slot].at[pl.ds(0, L)], sems[slot])
        pltpu.async_copy(table_hbm.at[idx_v.at[2 * c + 1]],
                         stage_v.at[slot].at[pl.ds(L, L)], sems[slot])

    def wait(c, slot):
        pltpu.make_async_copy(table_hbm.at[idx_v.at[2 * c]],
                              stage_v.at[slot].at[pl.ds(0, L)],
                              sems[slot]).wait()
        pltpu.make_async_copy(table_hbm.at[idx_v.at[2 * c + 1]],
                              stage_v.at[slot].at[pl.ds(L, L)],
                              sems[slot]).wait()

    def compute(c, slot, sq_acc):
        stage = stage_v.at[slot]
        rows = rows_v.at[slot]
        sql = [zero, zero, zero, zero]
        for r in range(CR):
            e0 = stage[r, pl.ds(0, 16)]
            e1 = stage[r, pl.ds(16, 16)]
            rows[r, pl.ds(0, 16)] = e0
            rows[r, pl.ds(16, 16)] = e1
            sql[r % 4] = sql[r % 4] + (e0 * e0 + e1 * e1)
        sq_acc = sq_acc + (sql[0] + sql[1]) + (sql[2] + sql[3])
        for bi in range(CB):
            bj = c * CB + bi
            acc_lo = [zero, zero, zero, zero]
            acc_hi = [zero, zero, zero, zero]
            u0 = u_v[bj, 0, pl.ds(0, 16)]
            u1 = u_v[bj, 0, pl.ds(16, 16)]
            for d in range(D):
                u_d = u0[d] if d < 16 else u1[d - 16]
                d_vec = jnp.full((16,), d, jnp.int32)
                for g in range(4):
                    rid = bi * L + g * 16 + lane
                    e = plsc.load_gather(rows, [rid, d_vec])
                    if d < 16:
                        acc_lo[g] = acc_lo[g] + u_d * e
                    else:
                        acc_hi[g] = acc_hi[g] + u_d * e
            for g in range(4):
                off = c * CR + bi * L + g * 16
                pred_v[pl.ds(off, 16)] = acc_lo[g] + acc_hi[g]
        return sq_acc

    for k in range(NBUF):
        start(k, k)

    def grp_body(p, sq_acc):
        for j in range(NBUF):
            c = NBUF * p + j
            wait(c, j)
            sq_acc = compute(c, j, sq_acc)

            @pl.when(c + NBUF < NCHUNK)
            def _():
                start(c + NBUF, j)

        return sq_acc

    sq = lax.fori_loop(0, NCHUNK // NBUF, grp_body, zero)

    def u_body(i, us):
        u0 = u_v[i, 0, pl.ds(0, 16)]
        u1 = u_v[i, 0, pl.ds(16, 16)]
        return us + u0 * u0 + u1 * u1

    us = lax.fori_loop(0, BW, u_body, zero)

    gsq_v[...] = sq
    usq_v[...] = us
    pltpu.sync_copy(pred_v.at[pl.ds(0, RW)], pred_hbm.at[pl.ds(wid * RW, RW)])
    pltpu.sync_copy(gsq_v, gsq_hbm.at[wid])
    pltpu.sync_copy(usq_v, usq_hbm.at[wid])


@functools.cache
def _sc_gather_dot():
    mesh = plsc.VectorSubcoreMesh(core_axis_name="c", subcore_axis_name="s",
                                  num_cores=NC, num_subcores=NS)
    return pl.kernel(
        _sc_body,
        out_type=(
            jax.ShapeDtypeStruct((B * L,), jnp.float32),
            jax.ShapeDtypeStruct((NW, LN), jnp.float32),
            jax.ShapeDtypeStruct((NW, LN), jnp.float32),
        ),
        mesh=mesh,
        compiler_params=pltpu.CompilerParams(needs_layout_passes=False,
                                             use_tc_tiling_on_sc=False),
        scratch_types=[
            pltpu.VMEM((BW, L), jnp.int32),
            pltpu.VMEM((BW, 1, D), jnp.float32),
            pltpu.VMEM((NBUF, CR, D), jnp.float32),
            pltpu.VMEM((NBUF, RPAD, PITCH), jnp.float32),
            pltpu.VMEM((RW + 16,), jnp.float32),
            pltpu.VMEM((LN,), jnp.float32),
            pltpu.VMEM((LN,), jnp.float32),
        ] + [pltpu.SemaphoreType.DMA] * NBUF,
    )


def _tc_body(p_ref, y_ref, m_ref, gsq_ref, usq_ref, out_ref):
    x = p_ref[...]
    y = y_ref[...]
    m = m_ref[...]
    bce = jnp.maximum(x, 0.0) - x * y + jnp.log1p(jnp.exp(-jnp.abs(x)))
    err = jnp.sum(bce * m)
    gs = jnp.sum(gsq_ref[...])
    us = jnp.sum(usq_ref[...])
    out_ref[0, 0] = err + LAM * (jnp.sqrt(us) + jnp.sqrt(gs))


def kernel(user_embedding_update, item, labels, mdsk, item_embeddings):
    idx = item.astype(jnp.int32).reshape(NW, BW, L)
    pred, gsq, usq = _sc_gather_dot()(item_embeddings, idx, user_embedding_update)

    p = pred.reshape(1600, 128)
    y = labels.reshape(1600, 128)
    m = mdsk.reshape(1600, 128)
    out = pl.pallas_call(
        _tc_body,
        out_shape=jax.ShapeDtypeStruct((1, 1), jnp.float32),
        out_specs=pl.BlockSpec(memory_space=pltpu.SMEM),
    )(p, y, m, gsq, usq)
    return out.reshape(())


# speedup vs baseline: 1.0060x; 1.0060x over previous
import functools

import jax
import jax.numpy as jnp
from jax import lax
from jax.experimental import pallas as pl
from jax.experimental.pallas import tpu as pltpu
from jax.experimental.pallas import tpu_sc as plsc

B, L, D, V = 4096, 50, 32, 100001
LAM = 0.01

NC, NS, LN = 2, 16, 16
NW = NC * NS
BW = B // NW
RW = BW * L
CB = 2
CR = CB * L
NCHUNK = BW // CB
RPAD = 128
NBUF = 2
PITCH = 40

def _sc_body(table_hbm, idx_hbm, u_hbm, pred_hbm, gsq_hbm, usq_hbm,
             idx_v, u_v, stage_v, rows_v, pred_v, gsq_v, usq_v, *sems):
    wid = lax.axis_index("s") * NC + lax.axis_index("c")
    pltpu.sync_copy(idx_hbm.at[wid], idx_v)
    pltpu.sync_copy(u_hbm.at[pl.ds(wid * BW, BW)], u_v)

    lane = lax.iota(jnp.int32, 16)
    zero = jnp.zeros((16,), jnp.float32)

    def start(c, slot):
        pltpu.async_copy(table_hbm.at[idx_v.at[2 * c]],
                         stage_v.at[slot].at[pl.ds(0, L)], sems[slot])
        pltpu.async_copy(table_hbm.at[idx_v.at[2 * c + 1]],
                         stage_v.at[slot].at[pl.ds(L, L)], sems[slot])

    def wait(c, slot):
        pltpu.make_async_copy(table_hbm.at[idx_v.at[2 * c]],
                              stage_v.at[slot].at[pl.ds(0, L)],
                              sems[slot]).wait()
        pltpu.make_async_copy(table_hbm.at[idx_v.at[2 * c + 1]],
                              stage_v.at[slot].at[pl.ds(L, L)],
                              sems[slot]).wait()

    def compute(c, slot, sq_acc):
        stage = stage_v.at[slot]
        rows = rows_v.at[slot]
        sql = [zero, zero, zero, zero]
        for r in range(CR):
            e0 = stage[r, pl.ds(0, 16)]
            e1 = stage[r, pl.ds(16, 16)]
            rows[r, pl.ds(0, 16)] = e0
            rows[r, pl.ds(16, 16)] = e1
            sql[r % 4] = sql[r % 4] + (e0 * e0 + e1 * e1)
        sq_acc = sq_acc + (sql[0] + sql[1]) + (sql[2] + sql[3])
        for bi in range(CB):
            bj = c * CB + bi
            acc_lo = [zero, zero, zero, zero]
            acc_hi = [zero, zero, zero, zero]
            u0 = u_v[bj, 0, pl.ds(0, 16)]
            u1 = u_v[bj, 0, pl.ds(16, 16)]
            for d in range(D):
                u_d = u0[d] if d < 16 else u1[d - 16]
                d_vec = jnp.full((16,), d, jnp.int32)
                for g in range(4):
                    rid = bi * L + g * 16 + lane
                    e = plsc.load_gather(rows, [rid, d_vec])
                    if d < 16:
                        acc_lo[g] = acc_lo[g] + u_d * e
                    else:
                        acc_hi[g] = acc_hi[g] + u_d * e
            for g in range(4):
                off = c * CR + bi * L + g * 16
                pred_v[pl.ds(off, 16)] = acc_lo[g] + acc_hi[g]
        return sq_acc

    for k in range(NBUF):
        start(k, k)

    def grp_body(p, sq_acc):
        for j in range(NBUF):
            c = NBUF * p + j
            wait(c, j)
            sq_acc = compute(c, j, sq_acc)

            @pl.when(c + NBUF < NCHUNK)
            def _():
                start(c + NBUF, j)

        return sq_acc

    sq = lax.fori_loop(0, NCHUNK // NBUF, grp_body, zero)

    def u_body(i, us):
        u0 = u_v[i, 0, pl.ds(0, 16)]
        u1 = u_v[i, 0, pl.ds(16, 16)]
        return us + u0 * u0 + u1 * u1

    us = lax.fori_loop(0, BW, u_body, zero)

    gsq_v[...] = sq
    usq_v[...] = us
    pltpu.sync_copy(pred_v.at[pl.ds(0, RW)], pred_hbm.at[pl.ds(wid * RW, RW)])
    pltpu.sync_copy(gsq_v, gsq_hbm.at[wid])
    pltpu.sync_copy(usq_v, usq_hbm.at[wid])


@functools.cache
def _sc_gather_dot():
    mesh = plsc.VectorSubcoreMesh(core_axis_name="c", subcore_axis_name="s",
                                  num_cores=NC, num_subcores=NS)
    return pl.kernel(
        _sc_body,
        out_type=(
            jax.ShapeDtypeStruct((B * L,), jnp.float32),
            jax.ShapeDtypeStruct((NW, LN), jnp.float32),
            jax.ShapeDtypeStruct((NW, LN), jnp.float32),
        ),
        mesh=mesh,
        compiler_params=pltpu.CompilerParams(needs_layout_passes=False,
                                             use_tc_tiling_on_sc=False),
        scratch_types=[
            pltpu.VMEM((BW, L), jnp.int32),
            pltpu.VMEM((BW, 1, D), jnp.float32),
            pltpu.VMEM((NBUF, CR, D), jnp.float32),
            pltpu.VMEM((NBUF, RPAD, PITCH), jnp.float32),
            pltpu.VMEM((RW + 16,), jnp.float32),
            pltpu.VMEM((LN,), jnp.float32),
            pltpu.VMEM((LN,), jnp.float32),
        ] + [pltpu.SemaphoreType.DMA] * NBUF,
    )


def _tc_body(p_ref, y_ref, m_ref, gsq_ref, usq_ref, out_ref):
    x = p_ref[...]
    y = y_ref[...]
    m = m_ref[...]
    bce = jnp.maximum(x, 0.0) - x * y + jnp.log1p(jnp.exp(-jnp.abs(x)))
    err = jnp.sum(bce * m)
    gs = jnp.sum(gsq_ref[...])
    us = jnp.sum(usq_ref[...])
    out_ref[0, 0] = err + LAM * (jnp.sqrt(us) + jnp.sqrt(gs))


def kernel(user_embedding_update, item, labels, mdsk, item_embeddings):
    idx = item.astype(jnp.int32).reshape(NW, BW, L)
    pred, gsq, usq = _sc_gather_dot()(item_embeddings, idx, user_embedding_update)

    p = pred.reshape(1600, 128)
    y = labels.reshape(1600, 128)
    m = mdsk.reshape(1600, 128)
    out = pl.pallas_call(
        _tc_body,
        out_shape=jax.ShapeDtypeStruct((1, 1), jnp.float32),
        out_specs=pl.BlockSpec(memory_space=pltpu.SMEM),
    )(p, y, m, gsq, usq)
    return out.reshape(())


# speedup vs baseline: 1.4835x; 1.4747x over previous
import functools

import jax
import jax.numpy as jnp
from jax import lax
from jax.experimental import pallas as pl
from jax.experimental.pallas import tpu as pltpu
from jax.experimental.pallas import tpu_sc as plsc

B, L, D, V = 4096, 50, 32, 100001
LAM = 0.01

NC, NS, LN = 2, 16, 16
NW = NC * NS
BW = B // NW
RW = BW * L
CB = 2
CR = CB * L
NCHUNK = BW // CB
RPAD = 128
NBUF = 2
PITCH = 40

def _sc_body(table_hbm, idx_hbm, u_hbm, pred_hbm, gsq_hbm, usq_hbm,
             idx_v, u_v, stage_v, pred_v, gsq_v, usq_v, *sems):
    wid = lax.axis_index("s") * NC + lax.axis_index("c")
    pltpu.sync_copy(idx_hbm.at[wid], idx_v)
    pltpu.sync_copy(u_hbm.at[pl.ds(wid * BW, BW)], u_v)

    lane = lax.iota(jnp.int32, 16)
    zero = jnp.zeros((16,), jnp.float32)

    def start(c, slot):
        pltpu.async_copy(table_hbm.at[idx_v.at[2 * c]],
                         stage_v.at[slot].at[pl.ds(0, L)], sems[slot])
        pltpu.async_copy(table_hbm.at[idx_v.at[2 * c + 1]],
                         stage_v.at[slot].at[pl.ds(L, L)], sems[slot])

    def wait(c, slot):
        pltpu.make_async_copy(table_hbm.at[idx_v.at[2 * c]],
                              stage_v.at[slot].at[pl.ds(0, L)],
                              sems[slot]).wait()
        pltpu.make_async_copy(table_hbm.at[idx_v.at[2 * c + 1]],
                              stage_v.at[slot].at[pl.ds(L, L)],
                              sems[slot]).wait()

    def compute(c, slot, sq_acc):
        stage = stage_v.at[slot]
        sql = [zero, zero, zero, zero]
        bvec = zero
        for bi in range(CB):
            bj = c * CB + bi
            u0v = u_v[bj, 0, pl.ds(0, 16)]
            u1v = u_v[bj, 0, pl.ds(16, 16)]
            for q in range(4):
                n = 16 if q < 3 else L - 48
                for t in range(n):
                    r = bi * L + q * 16 + t
                    e0 = stage[r, pl.ds(0, 16)]
                    e1 = stage[r, pl.ds(16, 16)]
                    sql[r % 4] = sql[r % 4] + (e0 * e0 + e1 * e1)
                    p = e0 * u0v + e1 * u1v
                    cs = plsc.cumsum(p)
                    bvec = jnp.where(lane == t, cs[15], bvec)
                pred_v[pl.ds(c * CR + bi * L + q * 16, 16)] = bvec
        return sq_acc + (sql[0] + sql[1]) + (sql[2] + sql[3])

    for k in range(NBUF):
        start(k, k)

    def grp_body(p, sq_acc):
        for j in range(NBUF):
            c = NBUF * p + j
            wait(c, j)
            sq_acc = compute(c, j, sq_acc)

            @pl.when(c + NBUF < NCHUNK)
            def _():
                start(c + NBUF, j)

        return sq_acc

    sq = lax.fori_loop(0, NCHUNK // NBUF, grp_body, zero)

    def u_body(i, us):
        u0 = u_v[i, 0, pl.ds(0, 16)]
        u1 = u_v[i, 0, pl.ds(16, 16)]
        return us + u0 * u0 + u1 * u1

    us = lax.fori_loop(0, BW, u_body, zero)

    gsq_v[...] = sq
    usq_v[...] = us
    pltpu.sync_copy(pred_v.at[pl.ds(0, RW)], pred_hbm.at[pl.ds(wid * RW, RW)])
    pltpu.sync_copy(gsq_v, gsq_hbm.at[wid])
    pltpu.sync_copy(usq_v, usq_hbm.at[wid])


@functools.cache
def _sc_gather_dot():
    mesh = plsc.VectorSubcoreMesh(core_axis_name="c", subcore_axis_name="s",
                                  num_cores=NC, num_subcores=NS)
    return pl.kernel(
        _sc_body,
        out_type=(
            jax.ShapeDtypeStruct((B * L,), jnp.float32),
            jax.ShapeDtypeStruct((NW, LN), jnp.float32),
            jax.ShapeDtypeStruct((NW, LN), jnp.float32),
        ),
        mesh=mesh,
        compiler_params=pltpu.CompilerParams(needs_layout_passes=False,
                                             use_tc_tiling_on_sc=False),
        scratch_types=[
            pltpu.VMEM((BW, L), jnp.int32),
            pltpu.VMEM((BW, 1, D), jnp.float32),
            pltpu.VMEM((NBUF, CR, D), jnp.float32),
            pltpu.VMEM((RW + 16,), jnp.float32),
            pltpu.VMEM((LN,), jnp.float32),
            pltpu.VMEM((LN,), jnp.float32),
        ] + [pltpu.SemaphoreType.DMA] * NBUF,
    )


def _tc_body(p_ref, y_ref, m_ref, gsq_ref, usq_ref, out_ref):
    x = p_ref[...]
    y = y_ref[...]
    m = m_ref[...]
    bce = jnp.maximum(x, 0.0) - x * y + jnp.log1p(jnp.exp(-jnp.abs(x)))
    err = jnp.sum(bce * m)
    gs = jnp.sum(gsq_ref[...])
    us = jnp.sum(usq_ref[...])
    out_ref[0, 0] = err + LAM * (jnp.sqrt(us) + jnp.sqrt(gs))


def kernel(user_embedding_update, item, labels, mdsk, item_embeddings):
    idx = item.astype(jnp.int32).reshape(NW, BW, L)
    pred, gsq, usq = _sc_gather_dot()(item_embeddings, idx, user_embedding_update)

    p = pred.reshape(1600, 128)
    y = labels.reshape(1600, 128)
    m = mdsk.reshape(1600, 128)
    out = pl.pallas_call(
        _tc_body,
        out_shape=jax.ShapeDtypeStruct((1, 1), jnp.float32),
        out_specs=pl.BlockSpec(memory_space=pltpu.SMEM),
    )(p, y, m, gsq, usq)
    return out.reshape(())
